# baseline (speedup 1.0000x reference)
import jax
import jax.numpy as jnp
from jax import lax
from jax.experimental import pallas as pl
from jax.experimental.pallas import tpu as pltpu
from jax.experimental.pallas import tpu_sc as plsc

BATCH = 16384
F = 64
WIDE = 2 * F
NC = 2
NS = 16
NW = NC * NS
BPW = BATCH // NW
CHUNK = 128
NCHUNK = BPW // CHUNK
NBUF = 3
GROUPS = CHUNK // 16


def _mf_body(uw_hbm, iw_hbm, uoff_hbm, ioff_hbm, utw_hbm, itw_hbm,
             wv_hbm, bv_hbm, out_hbm,
             uidx_v, iidx_v, uoff_v, ioff_v, ubuf, ibuf, wv_v, bv_v, out_v,
             *sems):
    wid = lax.axis_index("s") * NC + lax.axis_index("c")
    lanes = lax.iota(jnp.int32, 16)

    pltpu.sync_copy(uw_hbm.at[wid], uidx_v)
    pltpu.sync_copy(iw_hbm.at[wid], iidx_v)
    pltpu.sync_copy(uoff_hbm.at[wid], uoff_v)
    pltpu.sync_copy(ioff_hbm.at[wid], ioff_v)
    pltpu.sync_copy(wv_hbm, wv_v)
    pltpu.sync_copy(bv_hbm, bv_v)

    def fire(t):
        b = t % NBUF
        cu = pltpu.async_copy(utw_hbm.at[uidx_v.at[t]], ubuf.at[b], sems[b])
        ci = pltpu.async_copy(itw_hbm.at[iidx_v.at[t]], ibuf.at[b], sems[NBUF + b])
        return cu, ci

    inflight = {}
    for t in range(min(NBUF, NCHUNK)):
        inflight[t] = fire(t)

    for t in range(NCHUNK):
        cu, ci = inflight.pop(t)
        cu.wait()
        ci.wait()
        b = t % NBUF
        ub = ubuf.at[b]
        ib = ibuf.at[b]
        w0 = wv_v[0]
        w1 = wv_v[1]
        w2 = wv_v[2]
        w3 = wv_v[3]
        ws = (w0, w1, w2, w3)

        def group(g, carry):
            base = t * CHUNK + g * 16
            uoffs = uoff_v[pl.ds(base, 16)]
            ioffs = ioff_v[pl.ds(base, 16)]
            out = bv_v[...]
            for j in range(16):
                r = g * 16 + j
                ou = uoffs[j]
                oi = ioffs[j]
                acc = (ub[r, pl.ds(ou, 16)] * ib[r, pl.ds(oi, 16)]) * ws[0]
                for c in range(1, 4):
                    acc = acc + (ub[r, pl.ds(ou + c * 16, 16)]
                                 * ib[r, pl.ds(oi + c * 16, 16)]) * ws[c]
                for sh in (8, 4, 2, 1):
                    acc = acc + jnp.take(acc, lanes ^ sh)
                out = jnp.where(lanes == j, out + acc, out)
            out_v[pl.ds(base, 16)] = 1.0 / (1.0 + jnp.exp(-out))
            return carry

        lax.fori_loop(0, GROUPS, group, 0)
        if t + NBUF < NCHUNK:
            inflight[t + NBUF] = fire(t + NBUF)

    pltpu.sync_copy(out_v, out_hbm.at[pl.ds(wid * BPW, BPW)])


@jax.jit
def kernel(user, item, user_table, item_table, W, b):
    user = user.astype(jnp.int32)
    item = item.astype(jnp.int32)
    uw = lax.shift_right_logical(user, 1).reshape(NW, NCHUNK, CHUNK)
    iw = lax.shift_right_logical(item, 1).reshape(NW, NCHUNK, CHUNK)
    uoff = ((user & 1) * F).reshape(NW, BPW)
    ioff = ((item & 1) * F).reshape(NW, BPW)
    utw = user_table.reshape(-1, WIDE)
    itw = item_table.reshape(-1, WIDE)
    wv = W.reshape(4, 16)
    bv = jnp.broadcast_to(b.reshape(1), (16,))

    mesh = plsc.VectorSubcoreMesh(core_axis_name="c", subcore_axis_name="s")
    run = pl.kernel(
        _mf_body,
        out_type=jax.ShapeDtypeStruct((BATCH,), jnp.float32),
        mesh=mesh,
        scratch_types=[
            pltpu.VMEM((NCHUNK, CHUNK), jnp.int32),
            pltpu.VMEM((NCHUNK, CHUNK), jnp.int32),
            pltpu.VMEM((BPW,), jnp.int32),
            pltpu.VMEM((BPW,), jnp.int32),
            pltpu.VMEM((NBUF, CHUNK, WIDE), jnp.float32),
            pltpu.VMEM((NBUF, CHUNK, WIDE), jnp.float32),
            pltpu.VMEM((4, 16), jnp.float32),
            pltpu.VMEM((16,), jnp.float32),
            pltpu.VMEM((BPW,), jnp.float32),
        ] + [pltpu.SemaphoreType.DMA] * (2 * NBUF),
    )
    return run(uw, iw, uoff, ioff, utw, itw, wv, bv)


# speedup vs baseline: 1.5660x; 1.5660x over previous
import jax
import jax.numpy as jnp
from jax import lax
from jax.experimental import pallas as pl
from jax.experimental.pallas import tpu as pltpu
from jax.experimental.pallas import tpu_sc as plsc

BATCH = 16384
F = 64
NC = 2
NS = 16
NW = NC * NS
BPW = BATCH // NW
G = 16
NGROUP = BPW // G


def _mf_body(user_hbm, item_hbm, ut_hbm, it_hbm, wv_hbm, bv_hbm, out_hbm,
             uidx_v, iidx_v, ubuf, ibuf, wv_v, bv_v, out_v,
             sem_u0, sem_u1, sem_i0, sem_i1):
    wid = lax.axis_index("s") * NC + lax.axis_index("c")
    base = wid * BPW
    lanes = lax.iota(jnp.int32, 16)

    pltpu.sync_copy(user_hbm.at[pl.ds(base, BPW)], uidx_v)
    pltpu.sync_copy(item_hbm.at[pl.ds(base, BPW)], iidx_v)
    pltpu.sync_copy(wv_hbm, wv_v)
    pltpu.sync_copy(bv_hbm, bv_v)

    usems = (sem_u0, sem_u1)
    isems = (sem_i0, sem_i1)

    def fire(g, par):
        uk = uidx_v[pl.ds(g * G, G)]
        ik = iidx_v[pl.ds(g * G, G)]
        for j in range(G):
            pltpu.async_copy(ut_hbm.at[uk[j]], ubuf.at[par].at[j], usems[par])
            pltpu.async_copy(it_hbm.at[ik[j]], ibuf.at[par].at[j], isems[par])

    def drain(par):
        pltpu.make_async_copy(ut_hbm.at[pl.ds(0, G)], ubuf.at[par], usems[par]).wait()
        pltpu.make_async_copy(it_hbm.at[pl.ds(0, G)], ibuf.at[par], isems[par]).wait()

    fire(0, 0)

    w0 = wv_v[0]
    w1 = wv_v[1]
    w2 = wv_v[2]
    w3 = wv_v[3]
    ws = (w0, w1, w2, w3)

    def compute(g, par):
        drain(par)
        ub = ubuf.at[par]
        ib = ibuf.at[par]
        out = bv_v[...]
        for j in range(G):
            acc = (ub[j, pl.ds(0, 16)] * ib[j, pl.ds(0, 16)]) * ws[0]
            for c in range(1, 4):
                acc = acc + (ub[j, pl.ds(c * 16, 16)]
                             * ib[j, pl.ds(c * 16, 16)]) * ws[c]
            for sh in (8, 4, 2, 1):
                acc = acc + jnp.take(acc, lanes ^ sh)
            out = jnp.where(lanes == j, out + acc, out)
        out_v[pl.ds(g * G, G)] = 1.0 / (1.0 + jnp.exp(-out))

    def step(m, carry):
        g0 = 2 * m
        fire(g0 + 1, 1)
        compute(g0, 0)

        @pl.when(g0 + 2 < NGROUP)
        def _():
            fire(g0 + 2, 0)
        compute(g0 + 1, 1)
        return carry

    lax.fori_loop(0, NGROUP // 2, step, 0)
    pltpu.sync_copy(out_v, out_hbm.at[pl.ds(base, BPW)])


@jax.jit
def kernel(user, item, user_table, item_table, W, b):
    user = user.astype(jnp.int32)
    item = item.astype(jnp.int32)
    wv = W.reshape(4, 16)
    bv = jnp.broadcast_to(b.reshape(1), (16,))

    mesh = plsc.VectorSubcoreMesh(core_axis_name="c", subcore_axis_name="s")
    run = pl.kernel(
        _mf_body,
        out_type=jax.ShapeDtypeStruct((BATCH,), jnp.float32),
        mesh=mesh,
        scratch_types=[
            pltpu.VMEM((BPW,), jnp.int32),
            pltpu.VMEM((BPW,), jnp.int32),
            pltpu.VMEM((2, G, F), jnp.float32),
            pltpu.VMEM((2, G, F), jnp.float32),
            pltpu.VMEM((4, 16), jnp.float32),
            pltpu.VMEM((16,), jnp.float32),
            pltpu.VMEM((BPW,), jnp.float32),
        ] + [pltpu.SemaphoreType.DMA] * 4,
    )
    return run(user, item, user_table, item_table, wv, bv)


# speedup vs baseline: 2.7489x; 1.7554x over previous
import jax
import jax.numpy as jnp
from jax import lax
from jax.experimental import pallas as pl
from jax.experimental.pallas import tpu as pltpu
from jax.experimental.pallas import tpu_sc as plsc

BATCH = 16384
F = 64
NC = 2
NS = 16
NW = NC * NS
BPW = BATCH // NW
G = 16
NGROUP = BPW // G
SLOTS = 4
FU = 8


def _mf_body(user_hbm, item_hbm, utt_hbm, itt_hbm, wb_hbm, bv_hbm, out_hbm,
             uidx_v, iidx_v, ubuf, ibuf, wb_v, bv_v, out_v, *sems):
    wid = lax.axis_index("s") * NC + lax.axis_index("c")
    base = wid * BPW
    lanes = lax.iota(jnp.int32, 16)

    pltpu.sync_copy(user_hbm.at[pl.ds(base, BPW)], uidx_v)
    pltpu.sync_copy(item_hbm.at[pl.ds(base, BPW)], iidx_v)
    pltpu.sync_copy(wb_hbm, wb_v)
    pltpu.sync_copy(bv_hbm, bv_v)

    usems = sems[:SLOTS]
    isems = sems[SLOTS:]

    def fire(slot, ku, ki):
        qu = pl.multiple_of((ku >> 7) << 7, 128)
        qi = pl.multiple_of((ki >> 7) << 7, 128)
        pltpu.async_copy(utt_hbm.at[:, pl.ds(qu, 128)], ubuf.at[slot], usems[slot])
        pltpu.async_copy(itt_hbm.at[:, pl.ds(qi, 128)], ibuf.at[slot], isems[slot])

    def drain(slot):
        pltpu.make_async_copy(utt_hbm.at[:, pl.ds(0, 128)], ubuf.at[slot], usems[slot]).wait()
        pltpu.make_async_copy(itt_hbm.at[:, pl.ds(0, 128)], ibuf.at[slot], isems[slot]).wait()

    kv0u = uidx_v[pl.ds(0, 16)]
    kv0i = iidx_v[pl.ds(0, 16)]
    for j in range(SLOTS):
        fire(j, kv0u[j], kv0i[j])

    def group(g, carry):
        kvu = uidx_v[pl.ds(g * G, 16)]
        kvi = iidx_v[pl.ds(g * G, 16)]
        nxt = ((g + 1) & (NGROUP - 1)) * G
        kvu_n = uidx_v[pl.ds(nxt, 16)]
        kvi_n = iidx_v[pl.ds(nxt, 16)]
        out = bv_v[...]
        for j in range(G):
            slot = j % SLOTS
            ku = kvu[j]
            ki = kvi[j]
            jcu = (ku & 127) & ~15
            jci = (ki & 127) & ~15
            lu = jnp.full((16,), ku & 15, jnp.int32)
            li = jnp.full((16,), ki & 15, jnp.int32)
            drain(slot)
            ub = ubuf.at[slot]
            ib = ibuf.at[slot]

            def fbody(ff, acc):
                f0 = ff * FU
                for df in range(FU):
                    uf = jnp.take(ub[f0 + df, pl.ds(jcu, 16)], lu)
                    vf = jnp.take(ib[f0 + df, pl.ds(jci, 16)], li)
                    acc = acc + (uf * vf) * wb_v[f0 + df]
                return acc

            acc = lax.fori_loop(0, F // FU, fbody, jnp.zeros((16,), jnp.float32))

            if j < G - SLOTS:
                knu, kni = kvu[j + SLOTS], kvi[j + SLOTS]
            else:
                knu, kni = kvu_n[j + SLOTS - G], kvi_n[j + SLOTS - G]

            @pl.when(g * G + j + SLOTS < BPW)
            def _():
                fire(slot, knu, kni)

            out = jnp.where(lanes == j, out + acc, out)
        out_v[pl.ds(g * G, 16)] = 1.0 / (1.0 + jnp.exp(-out))
        return carry

    lax.fori_loop(0, NGROUP, group, 0)
    pltpu.sync_copy(out_v, out_hbm.at[pl.ds(base, BPW)])


@jax.jit
def kernel(user, item, user_table, item_table, W, b):
    user = user.astype(jnp.int32)
    item = item.astype(jnp.int32)
    utt = user_table.T
    itt = item_table.T
    wb = jnp.broadcast_to(W.reshape(F, 1), (F, 16))
    bv = jnp.broadcast_to(b.reshape(1), (16,))

    mesh = plsc.VectorSubcoreMesh(core_axis_name="c", subcore_axis_name="s")
    run = pl.kernel(
        _mf_body,
        out_type=jax.ShapeDtypeStruct((BATCH,), jnp.float32),
        mesh=mesh,
        scratch_types=[
            pltpu.VMEM((BPW,), jnp.int32),
            pltpu.VMEM((BPW,), jnp.int32),
            pltpu.VMEM((SLOTS, F, 128), jnp.float32),
            pltpu.VMEM((SLOTS, F, 128), jnp.float32),
            pltpu.VMEM((F, 16), jnp.float32),
            pltpu.VMEM((16,), jnp.float32),
            pltpu.VMEM((BPW,), jnp.float32),
        ] + [pltpu.SemaphoreType.DMA] * (2 * SLOTS),
    )
    return run(user, item, utt, itt, wb, bv)
